# Initial kernel scaffold; baseline (speedup 1.0000x reference)
import jax
import jax.numpy as jnp
from jax.experimental import pallas as pl


def kernel(node_subgraph, adj_row, adj_col, adj_val, feat_full, label_full, W0_0, W0_1, b0_0, b0_1, W1_0, W1_1, b1_0, b1_1, Wc, bc):
    raise NotImplementedError("write your pallas kernel here")


# speedup vs baseline: 2.6182x; 2.6182x over previous
import dataclasses
import functools

import jax
import jax.numpy as jnp
from jax import lax
from jax.experimental import pallas as pl
from jax.experimental.pallas import tpu as pltpu
from jax.experimental.pallas import tpu_sc as plsc

N_SUB = 10000
N_FULL = 50000
E = 320000
D = 128
NPAD = 10240
NC, NS = 2, 16
NW = NC * NS
CH = 80
LABP = 128

def _sc_compiler_params():
    cp = pltpu.CompilerParams()
    if "needs_layout_passes" in pltpu.CompilerParams.__dataclass_fields__:
        cp = dataclasses.replace(cp, needs_layout_passes=False)
    return cp


@functools.cache
def _mesh():
    return plsc.VectorSubcoreMesh(core_axis_name="c", subcore_axis_name="s",
                                  num_cores=NC, num_subcores=NS)


def _wid():
    return lax.axis_index("s") * NC + lax.axis_index("c")


def _gather_body(idx_hbm, feat_hbm, lab_hbm, feat_out, lab_out,
                 idxv, fbuf, lbuf, sem1, sem2):
    w = _wid()
    rows_per_tile = NPAD // NW

    @pl.loop(0, rows_per_tile // CH)
    def _chunk(c):
        base = w * rows_per_tile + c * CH
        pltpu.sync_copy(idx_hbm.at[pl.ds(base, CH)], idxv)
        cp1 = pltpu.async_copy(feat_hbm.at[idxv], fbuf, sem1)
        cp2 = pltpu.async_copy(lab_hbm.at[idxv], lbuf, sem2)
        cp1.wait()
        cp2.wait()
        pltpu.sync_copy(fbuf, feat_out.at[pl.ds(base, CH)])
        pltpu.sync_copy(lbuf, lab_out.at[pl.ds(base, CH)])


@functools.cache
def _gather_call():
    return pl.kernel(
        _gather_body,
        out_type=(jax.ShapeDtypeStruct((NPAD, D), jnp.float32),
                  jax.ShapeDtypeStruct((NPAD, LABP), jnp.float32)),
        mesh=_mesh(),
        scratch_types=[
            pltpu.VMEM((CH,), jnp.int32),
            pltpu.VMEM((CH, D), jnp.float32),
            pltpu.VMEM((CH, LABP), jnp.float32),
            pltpu.SemaphoreType.DMA,
            pltpu.SemaphoreType.DMA,
        ],
    )


def _zero_buf(rowsv):
    @pl.loop(0, CH)
    def _z(e):
        for j in range(D // 16):
            rowsv.at[e, pl.ds(j * 16, 16)][...] = jnp.zeros((16,), jnp.float32)


def _zero_acc(rowsv, acc):
    sid = lax.axis_index("s")
    rows_per_sub = NPAD // NS

    @pl.loop(0, rows_per_sub // CH)
    def _z(k):
        pltpu.sync_copy(rowsv, acc.at[pl.ds(sid * rows_per_sub + k * CH, CH)])


def _spmm_edges(x_hbm, row_hbm, col_hbm, val_hbm, colv, rowv, valv, rowsv, acc,
                sem, base, nchunk):
    @pl.loop(0, nchunk)
    def _chunk(c):
        off = base + c * CH
        pltpu.sync_copy(col_hbm.at[pl.ds(off, CH)], colv)
        pltpu.sync_copy(row_hbm.at[pl.ds(off, CH)], rowv)
        pltpu.sync_copy(val_hbm.at[pl.ds(off, CH)], valv)
        pltpu.async_copy(x_hbm.at[colv], rowsv, sem).wait()

        @pl.loop(0, CH)
        def _scale(e):
            vidx = jnp.full((16,), e, dtype=jnp.int32)
            v = plsc.load_gather(valv, [vidx])
            for j in range(D // 16):
                sl = pl.ds(j * 16, 16)
                rowsv.at[e, sl][...] = rowsv.at[e, sl][...] * v

        pltpu.sync_copy(rowsv, acc.at[rowv], add=True)


def _drain_acc(acc, out_hbm_core):
    sid = lax.axis_index("s")
    rows_per_sub = NPAD // NS

    @pl.loop(0, rows_per_sub // CH)
    def _d(k):
        r0 = sid * rows_per_sub + k * CH
        pltpu.sync_copy(acc.at[pl.ds(r0, CH)], out_hbm_core.at[pl.ds(r0, CH)])


_SPMM_SCRATCH = [
    pltpu.VMEM((CH,), jnp.int32),
    pltpu.VMEM((CH,), jnp.int32),
    pltpu.VMEM((CH,), jnp.float32),
    pltpu.VMEM((CH, D), jnp.float32),
    pltpu.VMEM_SHARED((NPAD, D), jnp.float32),
    pltpu.SemaphoreType.DMA,
]


def _spmm1_body(x_hbm, row_hbm, col_hbm, val_hbm, out_hbm,
                colv, rowv, valv, rowsv, acc, sem):
    cid = lax.axis_index("c")
    ept = E // NW
    _zero_buf(rowsv)
    _zero_acc(rowsv, acc)
    plsc.subcore_barrier()
    _spmm_edges(x_hbm, row_hbm, col_hbm, val_hbm, colv, rowv, valv, rowsv, acc,
                sem, _wid() * ept, ept // CH)
    plsc.subcore_barrier()
    _drain_acc(acc, out_hbm.at[cid])


@functools.cache
def _spmm1_call():
    return pl.kernel(
        _spmm1_body,
        out_type=jax.ShapeDtypeStruct((NC, NPAD, D), jnp.float32),
        mesh=_mesh(),
        scratch_types=_SPMM_SCRATCH,
        compiler_params=_sc_compiler_params(),
    )


def _spmm2_body(x0_hbm, x1_hbm, row_hbm, col_hbm, val_hbm, out_hbm,
                colv, rowv, valv, rowsv, acc, sem):
    cid = lax.axis_index("c")
    sid = lax.axis_index("s")
    ept = E // NS
    _zero_buf(rowsv)
    _zero_acc(rowsv, acc)
    plsc.subcore_barrier()

    @pl.when(cid == 0)
    def _c0():
        _spmm_edges(x0_hbm, row_hbm, col_hbm, val_hbm, colv, rowv, valv, rowsv,
                    acc, sem, sid * ept, ept // CH)

    @pl.when(cid == 1)
    def _c1():
        _spmm_edges(x1_hbm, row_hbm, col_hbm, val_hbm, colv, rowv, valv, rowsv,
                    acc, sem, sid * ept, ept // CH)

    plsc.subcore_barrier()
    _drain_acc(acc, out_hbm.at[cid])


@functools.cache
def _spmm2_call():
    return pl.kernel(
        _spmm2_body,
        out_type=jax.ShapeDtypeStruct((NC, NPAD, D), jnp.float32),
        mesh=_mesh(),
        scratch_types=_SPMM_SCRATCH,
        compiler_params=_sc_compiler_params(),
    )


_BM = 2048


def _layer1_body(feat_ref, ax0_ref, ax1_ref, w00_ref, w01_ref, b00_ref,
                 b01_ref, p0_ref, p1_ref):
    f = feat_ref[...]
    ax = ax0_ref[...] + ax1_ref[...]
    p0_ref[...] = jnp.maximum(
        jnp.dot(f, w00_ref[...], preferred_element_type=jnp.float32)
        + b00_ref[...], 0.0)
    p1_ref[...] = jnp.maximum(
        jnp.dot(ax, w01_ref[...], preferred_element_type=jnp.float32)
        + b01_ref[...], 0.0)


def _layer1_call(feat, ax0, ax1, w00, w01, b00, b01):
    row_spec = pl.BlockSpec((_BM, D), lambda i: (i, 0))
    full = pl.BlockSpec((D, D), lambda i: (0, 0))
    bias = pl.BlockSpec((1, D), lambda i: (0, 0))
    return pl.pallas_call(
        _layer1_body,
        grid=(NPAD // _BM,),
        in_specs=[row_spec, row_spec, row_spec, full, full, bias, bias],
        out_specs=[row_spec, row_spec],
        out_shape=[jax.ShapeDtypeStruct((NPAD, D), jnp.float32)] * 2,
    )(feat, ax0, ax1, w00, w01, b00, b01)


def _layer2_body(p0_ref, p1_ref, a0_ref, a1_ref, w10a_ref, w10b_ref,
                 w11a_ref, w11b_ref, b10_ref, b11_ref, wc0_ref, wc1_ref,
                 bc_ref, pred_ref):
    p0, p1 = p0_ref[...], p1_ref[...]
    q0 = jnp.maximum(
        jnp.dot(p0, w10a_ref[...], preferred_element_type=jnp.float32)
        + jnp.dot(p1, w10b_ref[...], preferred_element_type=jnp.float32)
        + b10_ref[...], 0.0)
    q1 = jnp.maximum(
        jnp.dot(a0_ref[...], w11a_ref[...], preferred_element_type=jnp.float32)
        + jnp.dot(a1_ref[...], w11b_ref[...], preferred_element_type=jnp.float32)
        + b11_ref[...], 0.0)
    s = jnp.sum(q0 * q0, axis=1, keepdims=True) + jnp.sum(q1 * q1, axis=1,
                                                          keepdims=True)
    norm = jnp.maximum(jnp.sqrt(s), 1e-12)
    pred_ref[...] = (
        jnp.dot(q0, wc0_ref[...], preferred_element_type=jnp.float32)
        + jnp.dot(q1, wc1_ref[...], preferred_element_type=jnp.float32)
    ) / norm + bc_ref[...]


def _layer2_call(p0, p1, a0, a1, w10a, w10b, w11a, w11b, b10, b11, wc0, wc1,
                 bcp):
    row_spec = pl.BlockSpec((_BM, D), lambda i: (i, 0))
    full = pl.BlockSpec((D, D), lambda i: (0, 0))
    bias = pl.BlockSpec((1, D), lambda i: (0, 0))
    wc_spec = pl.BlockSpec((D, LABP), lambda i: (0, 0))
    bc_spec = pl.BlockSpec((1, LABP), lambda i: (0, 0))
    return pl.pallas_call(
        _layer2_body,
        grid=(NPAD // _BM,),
        in_specs=[row_spec, row_spec, row_spec, row_spec, full, full, full,
                  full, bias, bias, wc_spec, wc_spec, bc_spec],
        out_specs=pl.BlockSpec((_BM, LABP), lambda i: (i, 0)),
        out_shape=jax.ShapeDtypeStruct((NPAD, LABP), jnp.float32),
    )(p0, p1, a0, a1, w10a, w10b, w11a, w11b, b10, b11, wc0, wc1, bcp)


def _argmax_body(lab_ref, o_ref):
    o_ref[...] = jnp.argmax(lab_ref[...], axis=1).astype(jnp.int32)[:, None]


def _argmax_call(labg):
    return pl.pallas_call(
        _argmax_body,
        grid=(NPAD // _BM,),
        in_specs=[pl.BlockSpec((_BM, LABP), lambda i: (i, 0))],
        out_specs=pl.BlockSpec((_BM, 1), lambda i: (i, 0)),
        out_shape=jax.ShapeDtypeStruct((NPAD, 1), jnp.int32),
    )(labg)


def kernel(node_subgraph, adj_row, adj_col, adj_val, feat_full, label_full,
           W0_0, W0_1, b0_0, b0_1, W1_0, W1_1, b1_0, b1_1, Wc, bc):
    C = Wc.shape[1]
    ns_pad = jnp.concatenate(
        [node_subgraph, jnp.zeros((NPAD - N_SUB,), jnp.int32)])
    lab_pad = jnp.pad(label_full, ((0, 0), (0, LABP - C)),
                      constant_values=-1e30)

    featg, labg = _gather_call()(ns_pad, feat_full, lab_pad)

    ax = _spmm1_call()(featg, adj_row, adj_col, adj_val)
    p0, p1 = _layer1_call(featg, ax[0], ax[1], W0_0, W0_1,
                          b0_0[None, :], b0_1[None, :])
    ah = _spmm2_call()(p0, p1, adj_row, adj_col, adj_val)
    pred_pad = _layer2_call(
        p0, p1, ah[0], ah[1],
        W1_0[:D], W1_0[D:], W1_1[:D], W1_1[D:],
        b1_0[None, :], b1_1[None, :],
        jnp.pad(Wc[:D], ((0, 0), (0, LABP - C))),
        jnp.pad(Wc[D:], ((0, 0), (0, LABP - C))),
        jnp.pad(bc, (0, LABP - C))[None, :])
    conv = _argmax_call(labg)

    return (pred_pad[:N_SUB, :C], labg[:N_SUB, :C], conv[:N_SUB, 0])


# speedup vs baseline: 5.8095x; 2.2189x over previous
import dataclasses
import functools

import jax
import jax.numpy as jnp
from jax import lax
from jax.experimental import pallas as pl
from jax.experimental.pallas import tpu as pltpu
from jax.experimental.pallas import tpu_sc as plsc

N_SUB = 10000
N_FULL = 50000
E = 320000
D = 128
NPAD = 10240
NC, NS = 2, 16
NW = NC * NS
CH = 80
LABP = 128

def _sc_compiler_params():
    cp = pltpu.CompilerParams()
    if "needs_layout_passes" in pltpu.CompilerParams.__dataclass_fields__:
        cp = dataclasses.replace(cp, needs_layout_passes=False)
    return cp


@functools.cache
def _mesh():
    return plsc.VectorSubcoreMesh(core_axis_name="c", subcore_axis_name="s",
                                  num_cores=NC, num_subcores=NS)


def _wid():
    return lax.axis_index("s") * NC + lax.axis_index("c")


def _gather_body(idx_hbm, feat_hbm, lab_hbm, feat_out, lab_out,
                 idxv, fbuf, lbuf, sem1, sem2):
    w = _wid()
    rows_per_tile = NPAD // NW

    @pl.loop(0, rows_per_tile // CH)
    def _chunk(c):
        base = w * rows_per_tile + c * CH
        pltpu.sync_copy(idx_hbm.at[pl.ds(base, CH)], idxv)
        cp1 = pltpu.async_copy(feat_hbm.at[idxv], fbuf, sem1)
        cp2 = pltpu.async_copy(lab_hbm.at[idxv], lbuf, sem2)
        cp1.wait()
        cp2.wait()
        pltpu.sync_copy(fbuf, feat_out.at[pl.ds(base, CH)])
        pltpu.sync_copy(lbuf, lab_out.at[pl.ds(base, CH)])


@functools.cache
def _gather_call():
    return pl.kernel(
        _gather_body,
        out_type=(jax.ShapeDtypeStruct((NPAD, D), jnp.float32),
                  jax.ShapeDtypeStruct((NPAD, LABP), jnp.float32)),
        mesh=_mesh(),
        scratch_types=[
            pltpu.VMEM((CH,), jnp.int32),
            pltpu.VMEM((CH, D), jnp.float32),
            pltpu.VMEM((CH, LABP), jnp.float32),
            pltpu.SemaphoreType.DMA,
            pltpu.SemaphoreType.DMA,
        ],
    )


def _zero_buf(rowsv):
    @pl.loop(0, CH)
    def _z(e):
        for j in range(D // 16):
            rowsv.at[e, pl.ds(j * 16, 16)][...] = jnp.zeros((16,), jnp.float32)


def _zero_acc(rowsv, acc):
    sid = lax.axis_index("s")
    rows_per_sub = NPAD // NS

    @pl.loop(0, rows_per_sub // CH)
    def _z(k):
        pltpu.sync_copy(rowsv, acc.at[pl.ds(sid * rows_per_sub + k * CH, CH)])


def _issue_chunk(x_hbm, row_hbm, col_hbm, val_hbm, buf, off):
    colv, rowv, valv, rowsv, gsem, isem = buf
    c1 = pltpu.async_copy(col_hbm.at[pl.ds(off, CH)], colv, isem)
    c2 = pltpu.async_copy(row_hbm.at[pl.ds(off, CH)], rowv, isem)
    c3 = pltpu.async_copy(val_hbm.at[pl.ds(off, CH)], valv, isem)
    c1.wait()
    c2.wait()
    c3.wait()
    pltpu.async_copy(x_hbm.at[colv], rowsv, gsem)


def _process_chunk(x_hbm, row_hbm, col_hbm, val_hbm, acc, buf, prefetch_off,
                   nchunk_off_end):
    colv, rowv, valv, rowsv, gsem, isem = buf
    pltpu.make_async_copy(x_hbm.at[colv], rowsv, gsem).wait()

    @plsc.parallel_loop(0, CH, unroll=8)
    def _scale(e):
        vidx = jnp.full((16,), e, dtype=jnp.int32)
        v = plsc.load_gather(valv, [vidx])
        for j in range(D // 16):
            sl = pl.ds(j * 16, 16)
            rowsv.at[e, sl][...] = rowsv.at[e, sl][...] * v

    pltpu.sync_copy(rowsv, acc.at[rowv], add=True)
    if prefetch_off is not None:
        @pl.when(prefetch_off < nchunk_off_end)
        def _pf():
            _issue_chunk(x_hbm, row_hbm, col_hbm, val_hbm, buf, prefetch_off)


def _spmm_edges(x_hbm, row_hbm, col_hbm, val_hbm, bufs, acc, base, nchunk):
    end = base + nchunk * CH
    _issue_chunk(x_hbm, row_hbm, col_hbm, val_hbm, bufs[0], base)
    _issue_chunk(x_hbm, row_hbm, col_hbm, val_hbm, bufs[1], base + CH)
    plsc.subcore_barrier()

    @pl.loop(0, nchunk // 2)
    def _pair(i):
        for b in range(2):
            c = 2 * i + b
            _process_chunk(x_hbm, row_hbm, col_hbm, val_hbm, acc, bufs[b],
                           base + (c + 2) * CH, end)
    if nchunk % 2:
        _process_chunk(x_hbm, row_hbm, col_hbm, val_hbm, acc, bufs[0],
                       None, end)


def _drain_acc(acc, out_hbm_core):
    sid = lax.axis_index("s")
    rows_per_sub = NPAD // NS

    @pl.loop(0, rows_per_sub // CH)
    def _d(k):
        r0 = sid * rows_per_sub + k * CH
        pltpu.sync_copy(acc.at[pl.ds(r0, CH)], out_hbm_core.at[pl.ds(r0, CH)])


_SPMM_SCRATCH = [
    pltpu.VMEM((CH,), jnp.int32),
    pltpu.VMEM((CH,), jnp.int32),
    pltpu.VMEM((CH,), jnp.float32),
    pltpu.VMEM((CH, D), jnp.float32),
    pltpu.SemaphoreType.DMA,
    pltpu.SemaphoreType.DMA,
    pltpu.VMEM((CH,), jnp.int32),
    pltpu.VMEM((CH,), jnp.int32),
    pltpu.VMEM((CH,), jnp.float32),
    pltpu.VMEM((CH, D), jnp.float32),
    pltpu.SemaphoreType.DMA,
    pltpu.SemaphoreType.DMA,
    pltpu.VMEM((CH, D), jnp.float32),
    pltpu.VMEM_SHARED((NPAD, D), jnp.float32),
]


def _spmm1_body(x_hbm, row_hbm, col_hbm, val_hbm, out_hbm, *rest):
    bufs = (rest[0:6], rest[6:12])
    zbuf, acc = rest[12], rest[13]
    cid = lax.axis_index("c")
    ept = E // NW
    _zero_buf(zbuf)
    _zero_acc(zbuf, acc)
    _spmm_edges(x_hbm, row_hbm, col_hbm, val_hbm, bufs, acc,
                _wid() * ept, ept // CH)
    plsc.subcore_barrier()
    _drain_acc(acc, out_hbm.at[cid])


@functools.cache
def _spmm1_call():
    return pl.kernel(
        _spmm1_body,
        out_type=jax.ShapeDtypeStruct((NC, NPAD, D), jnp.float32),
        mesh=_mesh(),
        scratch_types=_SPMM_SCRATCH,
        compiler_params=_sc_compiler_params(),
    )


def _spmm2_body(x0_hbm, x1_hbm, row_hbm, col_hbm, val_hbm, out_hbm, *rest):
    bufs = (rest[0:6], rest[6:12])
    zbuf, acc = rest[12], rest[13]
    cid = lax.axis_index("c")
    sid = lax.axis_index("s")
    ept = E // NS
    _zero_buf(zbuf)
    _zero_acc(zbuf, acc)

    @pl.when(cid == 0)
    def _c0():
        _spmm_edges(x0_hbm, row_hbm, col_hbm, val_hbm, bufs, acc,
                    sid * ept, ept // CH)

    @pl.when(cid == 1)
    def _c1():
        _spmm_edges(x1_hbm, row_hbm, col_hbm, val_hbm, bufs, acc,
                    sid * ept, ept // CH)

    plsc.subcore_barrier()
    _drain_acc(acc, out_hbm.at[cid])


@functools.cache
def _spmm2_call():
    return pl.kernel(
        _spmm2_body,
        out_type=jax.ShapeDtypeStruct((NC, NPAD, D), jnp.float32),
        mesh=_mesh(),
        scratch_types=_SPMM_SCRATCH,
        compiler_params=_sc_compiler_params(),
    )


_BM = 2048


def _layer1_body(feat_ref, ax0_ref, ax1_ref, w00_ref, w01_ref, b00_ref,
                 b01_ref, p0_ref, p1_ref):
    f = feat_ref[...]
    ax = ax0_ref[...] + ax1_ref[...]
    p0_ref[...] = jnp.maximum(
        jnp.dot(f, w00_ref[...], preferred_element_type=jnp.float32)
        + b00_ref[...], 0.0)
    p1_ref[...] = jnp.maximum(
        jnp.dot(ax, w01_ref[...], preferred_element_type=jnp.float32)
        + b01_ref[...], 0.0)


def _layer1_call(feat, ax0, ax1, w00, w01, b00, b01):
    row_spec = pl.BlockSpec((_BM, D), lambda i: (i, 0))
    full = pl.BlockSpec((D, D), lambda i: (0, 0))
    bias = pl.BlockSpec((1, D), lambda i: (0, 0))
    return pl.pallas_call(
        _layer1_body,
        grid=(NPAD // _BM,),
        in_specs=[row_spec, row_spec, row_spec, full, full, bias, bias],
        out_specs=[row_spec, row_spec],
        out_shape=[jax.ShapeDtypeStruct((NPAD, D), jnp.float32)] * 2,
    )(feat, ax0, ax1, w00, w01, b00, b01)


def _layer2_body(p0_ref, p1_ref, a0_ref, a1_ref, w10a_ref, w10b_ref,
                 w11a_ref, w11b_ref, b10_ref, b11_ref, wc0_ref, wc1_ref,
                 bc_ref, pred_ref):
    p0, p1 = p0_ref[...], p1_ref[...]
    q0 = jnp.maximum(
        jnp.dot(p0, w10a_ref[...], preferred_element_type=jnp.float32)
        + jnp.dot(p1, w10b_ref[...], preferred_element_type=jnp.float32)
        + b10_ref[...], 0.0)
    q1 = jnp.maximum(
        jnp.dot(a0_ref[...], w11a_ref[...], preferred_element_type=jnp.float32)
        + jnp.dot(a1_ref[...], w11b_ref[...], preferred_element_type=jnp.float32)
        + b11_ref[...], 0.0)
    s = jnp.sum(q0 * q0, axis=1, keepdims=True) + jnp.sum(q1 * q1, axis=1,
                                                          keepdims=True)
    norm = jnp.maximum(jnp.sqrt(s), 1e-12)
    pred_ref[...] = (
        jnp.dot(q0, wc0_ref[...], preferred_element_type=jnp.float32)
        + jnp.dot(q1, wc1_ref[...], preferred_element_type=jnp.float32)
    ) / norm + bc_ref[...]


def _layer2_call(p0, p1, a0, a1, w10a, w10b, w11a, w11b, b10, b11, wc0, wc1,
                 bcp):
    row_spec = pl.BlockSpec((_BM, D), lambda i: (i, 0))
    full = pl.BlockSpec((D, D), lambda i: (0, 0))
    bias = pl.BlockSpec((1, D), lambda i: (0, 0))
    wc_spec = pl.BlockSpec((D, LABP), lambda i: (0, 0))
    bc_spec = pl.BlockSpec((1, LABP), lambda i: (0, 0))
    return pl.pallas_call(
        _layer2_body,
        grid=(NPAD // _BM,),
        in_specs=[row_spec, row_spec, row_spec, row_spec, full, full, full,
                  full, bias, bias, wc_spec, wc_spec, bc_spec],
        out_specs=pl.BlockSpec((_BM, LABP), lambda i: (i, 0)),
        out_shape=jax.ShapeDtypeStruct((NPAD, LABP), jnp.float32),
    )(p0, p1, a0, a1, w10a, w10b, w11a, w11b, b10, b11, wc0, wc1, bcp)


def _argmax_body(lab_ref, o_ref):
    x = lab_ref[...]
    m = jnp.max(x, axis=1, keepdims=True)
    col = jax.lax.broadcasted_iota(jnp.int32, x.shape, 1)
    idx = jnp.where(x == m, col, jnp.int32(LABP))
    o_ref[...] = jnp.min(idx, axis=1).astype(jnp.int32)[:, None]


def _argmax_call(labg):
    return pl.pallas_call(
        _argmax_body,
        grid=(NPAD // _BM,),
        in_specs=[pl.BlockSpec((_BM, LABP), lambda i: (i, 0))],
        out_specs=pl.BlockSpec((_BM, 1), lambda i: (i, 0)),
        out_shape=jax.ShapeDtypeStruct((NPAD, 1), jnp.int32),
    )(labg)


def kernel(node_subgraph, adj_row, adj_col, adj_val, feat_full, label_full,
           W0_0, W0_1, b0_0, b0_1, W1_0, W1_1, b1_0, b1_1, Wc, bc):
    C = Wc.shape[1]
    ns_pad = jnp.concatenate(
        [node_subgraph, jnp.zeros((NPAD - N_SUB,), jnp.int32)])
    lab_pad = jnp.pad(label_full, ((0, 0), (0, LABP - C)),
                      constant_values=-1e30)

    featg, labg = _gather_call()(ns_pad, feat_full, lab_pad)

    ax = _spmm1_call()(featg, adj_row, adj_col, adj_val)
    p0, p1 = _layer1_call(featg, ax[0], ax[1], W0_0, W0_1,
                          b0_0[None, :], b0_1[None, :])
    ah = _spmm2_call()(p0, p1, adj_row, adj_col, adj_val)
    pred_pad = _layer2_call(
        p0, p1, ah[0], ah[1],
        W1_0[:D], W1_0[D:], W1_1[:D], W1_1[D:],
        b1_0[None, :], b1_1[None, :],
        jnp.pad(Wc[:D], ((0, 0), (0, LABP - C))),
        jnp.pad(Wc[D:], ((0, 0), (0, LABP - C))),
        jnp.pad(bc, (0, LABP - C))[None, :])
    conv = _argmax_call(labg)

    return (pred_pad[:N_SUB, :C], labg[:N_SUB, :C], conv[:N_SUB, 0])


# speedup vs baseline: 7.1732x; 1.2347x over previous
import dataclasses
import functools

import jax
import jax.numpy as jnp
from jax import lax
from jax.experimental import pallas as pl
from jax.experimental.pallas import tpu as pltpu
from jax.experimental.pallas import tpu_sc as plsc

N_SUB = 10000
N_FULL = 50000
E = 320000
D = 128
NPAD = 10240
NC, NS = 2, 16
NW = NC * NS
CH = 80
LABP = 128

def _sc_compiler_params():
    cp = pltpu.CompilerParams()
    if "needs_layout_passes" in pltpu.CompilerParams.__dataclass_fields__:
        cp = dataclasses.replace(cp, needs_layout_passes=False)
    return cp


@functools.cache
def _mesh():
    return plsc.VectorSubcoreMesh(core_axis_name="c", subcore_axis_name="s",
                                  num_cores=NC, num_subcores=NS)


def _wid():
    return lax.axis_index("s") * NC + lax.axis_index("c")


def _gather_body(idx_hbm, feat_hbm, lab_hbm, feat_out, lab_out,
                 idxv, fbuf, lbuf, sem1, sem2):
    w = _wid()
    rows_per_tile = NPAD // NW

    @pl.loop(0, rows_per_tile // CH)
    def _chunk(c):
        base = w * rows_per_tile + c * CH
        pltpu.sync_copy(idx_hbm.at[pl.ds(base, CH)], idxv)
        cp1 = pltpu.async_copy(feat_hbm.at[idxv], fbuf, sem1)
        cp2 = pltpu.async_copy(lab_hbm.at[idxv], lbuf, sem2)
        cp1.wait()
        cp2.wait()
        pltpu.sync_copy(fbuf, feat_out.at[pl.ds(base, CH)])
        pltpu.sync_copy(lbuf, lab_out.at[pl.ds(base, CH)])


@functools.cache
def _gather_call():
    return pl.kernel(
        _gather_body,
        out_type=(jax.ShapeDtypeStruct((NPAD, D), jnp.float32),
                  jax.ShapeDtypeStruct((NPAD, LABP), jnp.float32)),
        mesh=_mesh(),
        scratch_types=[
            pltpu.VMEM((CH,), jnp.int32),
            pltpu.VMEM((CH, D), jnp.float32),
            pltpu.VMEM((CH, LABP), jnp.float32),
            pltpu.SemaphoreType.DMA,
            pltpu.SemaphoreType.DMA,
        ],
    )


def _zero_buf(rowsv):
    @pl.loop(0, CH)
    def _z(e):
        for j in range(D // 16):
            rowsv.at[e, pl.ds(j * 16, 16)][...] = jnp.zeros((16,), jnp.float32)


def _zero_acc(rowsv, acc):
    sid = lax.axis_index("s")
    rows_per_sub = NPAD // NS

    @pl.loop(0, rows_per_sub // CH)
    def _z(k):
        pltpu.sync_copy(rowsv, acc.at[pl.ds(sid * rows_per_sub + k * CH, CH)])


NBUF = 3
PFD = 2


def _scale_rows(ebuf, rowsv):
    @plsc.parallel_loop(0, CH, unroll=8)
    def _scale(e):
        vrow = jnp.full((16,), 2, dtype=jnp.int32)
        vidx = jnp.full((16,), e, dtype=jnp.int32)
        v = plsc.bitcast(plsc.load_gather(ebuf, [vrow, vidx]), jnp.float32)
        for j in range(D // 16):
            sl = pl.ds(j * 16, 16)
            rowsv.at[e, sl][...] = rowsv.at[e, sl][...] * v


def _process_chunk(x_hbm, ep_hbm, acc, bufs, c, b, pf_gc, gc_end):
    ebuf, rowsv, gsem, isem, ssem = bufs[b]
    pb = (b + PFD) % NBUF
    pebuf, prowsv, pgsem, pisem, pssem = bufs[pb]

    if pf_gc is not None:
        @pl.when(pf_gc < gc_end)
        def _pf_idx():
            @pl.when(c >= NBUF - PFD)
            def _w():
                pltpu.make_async_copy(prowsv, acc.at[pebuf.at[0]], pssem).wait()
            pltpu.async_copy(ep_hbm.at[pf_gc], pebuf, pisem)

    pltpu.make_async_copy(x_hbm.at[ebuf.at[1]], rowsv, gsem).wait()
    _scale_rows(ebuf, rowsv)
    pltpu.async_copy(rowsv, acc.at[ebuf.at[0]], ssem, add=True)

    if pf_gc is not None:
        @pl.when(pf_gc < gc_end)
        def _pf_gather():
            pltpu.make_async_copy(ep_hbm.at[pf_gc], pebuf, pisem).wait()
            pltpu.async_copy(x_hbm.at[pebuf.at[1]], prowsv, pgsem)


def _issue_first(x_hbm, ep_hbm, bufs, b, gc):
    ebuf, rowsv, gsem, isem, ssem = bufs[b]
    pltpu.sync_copy(ep_hbm.at[gc], ebuf)
    pltpu.async_copy(x_hbm.at[ebuf.at[1]], rowsv, gsem)


def _spmm_edges(x_hbm, ep_hbm, bufs, acc, cbase, nchunk):
    gc_end = cbase + nchunk
    for k in range(PFD):
        _issue_first(x_hbm, ep_hbm, bufs, k, cbase + k)
    plsc.subcore_barrier()

    @pl.loop(0, nchunk // NBUF)
    def _round(i):
        for b in range(NBUF):
            c = NBUF * i + b
            _process_chunk(x_hbm, ep_hbm, acc, bufs, c, b, cbase + c + PFD,
                           gc_end)
    for r in range(nchunk % NBUF):
        c = (nchunk // NBUF) * NBUF + r
        _process_chunk(x_hbm, ep_hbm, acc, bufs, c, c % NBUF, None, gc_end)

    for b in range(NBUF):
        ebuf, rowsv, gsem, isem, ssem = bufs[b]
        pltpu.make_async_copy(rowsv, acc.at[ebuf.at[0]], ssem).wait()


def _drain_acc(acc, out_hbm_core):
    sid = lax.axis_index("s")
    rows_per_sub = NPAD // NS

    @pl.loop(0, rows_per_sub // CH)
    def _d(k):
        r0 = sid * rows_per_sub + k * CH
        pltpu.sync_copy(acc.at[pl.ds(r0, CH)], out_hbm_core.at[pl.ds(r0, CH)])


_SPMM_SCRATCH = [
    s for _ in range(NBUF)
    for s in (pltpu.VMEM((3, CH), jnp.int32),
              pltpu.VMEM((CH, D), jnp.float32),
              pltpu.SemaphoreType.DMA,
              pltpu.SemaphoreType.DMA,
              pltpu.SemaphoreType.DMA)
] + [pltpu.VMEM_SHARED((NPAD, D), jnp.float32)]


def _split_scratch(rest):
    bufs = tuple(tuple(rest[5 * b:5 * b + 5]) for b in range(NBUF))
    return bufs, bufs[0][1], rest[5 * NBUF]


def _spmm1_body(x_hbm, ep_hbm, out_hbm, *rest):
    bufs, zbuf, acc = _split_scratch(rest)
    cid = lax.axis_index("c")
    cpt = (E // CH) // NW
    _zero_buf(zbuf)
    _zero_acc(zbuf, acc)
    _spmm_edges(x_hbm, ep_hbm, bufs, acc, _wid() * cpt, cpt)
    plsc.subcore_barrier()
    _drain_acc(acc, out_hbm.at[cid])


@functools.cache
def _spmm1_call():
    return pl.kernel(
        _spmm1_body,
        out_type=jax.ShapeDtypeStruct((NC, NPAD, D), jnp.float32),
        mesh=_mesh(),
        scratch_types=_SPMM_SCRATCH,
        compiler_params=_sc_compiler_params(),
    )


def _spmm2_body(x0_hbm, x1_hbm, ep_hbm, out_hbm, *rest):
    bufs, zbuf, acc = _split_scratch(rest)
    cid = lax.axis_index("c")
    sid = lax.axis_index("s")
    cpt = (E // CH) // NS
    _zero_buf(zbuf)
    _zero_acc(zbuf, acc)

    @pl.when(cid == 0)
    def _c0():
        _spmm_edges(x0_hbm, ep_hbm, bufs, acc, sid * cpt, cpt)

    @pl.when(cid == 1)
    def _c1():
        _spmm_edges(x1_hbm, ep_hbm, bufs, acc, sid * cpt, cpt)

    plsc.subcore_barrier()
    _drain_acc(acc, out_hbm.at[cid])


@functools.cache
def _spmm2_call():
    return pl.kernel(
        _spmm2_body,
        out_type=jax.ShapeDtypeStruct((NC, NPAD, D), jnp.float32),
        mesh=_mesh(),
        scratch_types=_SPMM_SCRATCH,
        compiler_params=_sc_compiler_params(),
    )


_BM = 2048


def _layer1_body(feat_ref, ax0_ref, ax1_ref, w00_ref, w01_ref, b00_ref,
                 b01_ref, p0_ref, p1_ref):
    f = feat_ref[...]
    ax = ax0_ref[...] + ax1_ref[...]
    p0_ref[...] = jnp.maximum(
        jnp.dot(f, w00_ref[...], preferred_element_type=jnp.float32,
                precision=jax.lax.Precision.HIGHEST)
        + b00_ref[...], 0.0)
    p1_ref[...] = jnp.maximum(
        jnp.dot(ax, w01_ref[...], preferred_element_type=jnp.float32,
                precision=jax.lax.Precision.HIGHEST)
        + b01_ref[...], 0.0)


def _layer1_call(feat, ax0, ax1, w00, w01, b00, b01):
    row_spec = pl.BlockSpec((_BM, D), lambda i: (i, 0))
    full = pl.BlockSpec((D, D), lambda i: (0, 0))
    bias = pl.BlockSpec((1, D), lambda i: (0, 0))
    return pl.pallas_call(
        _layer1_body,
        grid=(NPAD // _BM,),
        in_specs=[row_spec, row_spec, row_spec, full, full, bias, bias],
        out_specs=[row_spec, row_spec],
        out_shape=[jax.ShapeDtypeStruct((NPAD, D), jnp.float32)] * 2,
    )(feat, ax0, ax1, w00, w01, b00, b01)


def _layer2_body(p0_ref, p1_ref, a0_ref, a1_ref, w10a_ref, w10b_ref,
                 w11a_ref, w11b_ref, b10_ref, b11_ref, wc0_ref, wc1_ref,
                 bc_ref, pred_ref):
    p0, p1 = p0_ref[...], p1_ref[...]
    q0 = jnp.maximum(
        jnp.dot(p0, w10a_ref[...], preferred_element_type=jnp.float32,
                precision=jax.lax.Precision.HIGHEST)
        + jnp.dot(p1, w10b_ref[...], preferred_element_type=jnp.float32,
                precision=jax.lax.Precision.HIGHEST)
        + b10_ref[...], 0.0)
    q1 = jnp.maximum(
        jnp.dot(a0_ref[...], w11a_ref[...], preferred_element_type=jnp.float32,
                precision=jax.lax.Precision.HIGHEST)
        + jnp.dot(a1_ref[...], w11b_ref[...], preferred_element_type=jnp.float32,
                precision=jax.lax.Precision.HIGHEST)
        + b11_ref[...], 0.0)
    s = jnp.sum(q0 * q0, axis=1, keepdims=True) + jnp.sum(q1 * q1, axis=1,
                                                          keepdims=True)
    norm = jnp.maximum(jnp.sqrt(s), 1e-12)
    pred_ref[...] = (
        jnp.dot(q0, wc0_ref[...], preferred_element_type=jnp.float32,
                precision=jax.lax.Precision.HIGHEST)
        + jnp.dot(q1, wc1_ref[...], preferred_element_type=jnp.float32,
                precision=jax.lax.Precision.HIGHEST)
    ) / norm + bc_ref[...]


def _layer2_call(p0, p1, a0, a1, w10a, w10b, w11a, w11b, b10, b11, wc0, wc1,
                 bcp):
    row_spec = pl.BlockSpec((_BM, D), lambda i: (i, 0))
    full = pl.BlockSpec((D, D), lambda i: (0, 0))
    bias = pl.BlockSpec((1, D), lambda i: (0, 0))
    wc_spec = pl.BlockSpec((D, LABP), lambda i: (0, 0))
    bc_spec = pl.BlockSpec((1, LABP), lambda i: (0, 0))
    return pl.pallas_call(
        _layer2_body,
        grid=(NPAD // _BM,),
        in_specs=[row_spec, row_spec, row_spec, row_spec, full, full, full,
                  full, bias, bias, wc_spec, wc_spec, bc_spec],
        out_specs=pl.BlockSpec((_BM, LABP), lambda i: (i, 0)),
        out_shape=jax.ShapeDtypeStruct((NPAD, LABP), jnp.float32),
    )(p0, p1, a0, a1, w10a, w10b, w11a, w11b, b10, b11, wc0, wc1, bcp)


def _labpad_body(lab_ref, o_ref):
    o_ref[...] = jnp.concatenate(
        [lab_ref[...],
         jnp.full((lab_ref.shape[0], LABP - lab_ref.shape[1]), -1e30,
                  jnp.float32)], axis=1)


def _labpad_call(label_full):
    c = label_full.shape[1]
    bm = 2000
    return pl.pallas_call(
        _labpad_body,
        grid=(N_FULL // bm,),
        in_specs=[pl.BlockSpec((bm, c), lambda i: (i, 0))],
        out_specs=pl.BlockSpec((bm, LABP), lambda i: (i, 0)),
        out_shape=jax.ShapeDtypeStruct((N_FULL, LABP), jnp.float32),
    )(label_full)


def _argmax_body(lab_ref, o_ref):
    x = lab_ref[...]
    m = jnp.max(x, axis=1, keepdims=True)
    col = jax.lax.broadcasted_iota(jnp.int32, x.shape, 1)
    idx = jnp.where(x == m, col, jnp.int32(LABP))
    o_ref[...] = jnp.min(idx, axis=1).astype(jnp.int32)[:, None]


def _argmax_call(labg):
    return pl.pallas_call(
        _argmax_body,
        grid=(NPAD // _BM,),
        in_specs=[pl.BlockSpec((_BM, LABP), lambda i: (i, 0))],
        out_specs=pl.BlockSpec((_BM, 1), lambda i: (i, 0)),
        out_shape=jax.ShapeDtypeStruct((NPAD, 1), jnp.int32),
    )(labg)


def kernel(node_subgraph, adj_row, adj_col, adj_val, feat_full, label_full,
           W0_0, W0_1, b0_0, b0_1, W1_0, W1_1, b1_0, b1_1, Wc, bc):
    C = Wc.shape[1]
    ns_pad = jnp.concatenate(
        [node_subgraph, jnp.zeros((NPAD - N_SUB,), jnp.int32)])
    lab_pad = _labpad_call(label_full)
    epack = jnp.stack(
        [adj_row.reshape(E // CH, CH),
         adj_col.reshape(E // CH, CH),
         jax.lax.bitcast_convert_type(adj_val, jnp.int32).reshape(E // CH, CH)],
        axis=1)

    featg, labg = _gather_call()(ns_pad, feat_full, lab_pad)

    ax = _spmm1_call()(featg, epack)
    p0, p1 = _layer1_call(featg, ax[0], ax[1], W0_0, W0_1,
                          b0_0[None, :], b0_1[None, :])
    ah = _spmm2_call()(p0, p1, epack)
    pred_pad = _layer2_call(
        p0, p1, ah[0], ah[1],
        W1_0[:D], W1_0[D:], W1_1[:D], W1_1[D:],
        b1_0[None, :], b1_1[None, :],
        jnp.pad(Wc[:D], ((0, 0), (0, LABP - C))),
        jnp.pad(Wc[D:], ((0, 0), (0, LABP - C))),
        jnp.pad(bc, (0, LABP - C))[None, :])
    conv = _argmax_call(labg)

    return (pred_pad[:N_SUB, :C], labg[:N_SUB, :C], conv[:N_SUB, 0])


# speedup vs baseline: 7.5597x; 1.0539x over previous
import dataclasses
import functools

import jax
import jax.numpy as jnp
from jax import lax
from jax.experimental import pallas as pl
from jax.experimental.pallas import tpu as pltpu
from jax.experimental.pallas import tpu_sc as plsc

N_SUB = 10000
N_FULL = 50000
E = 320000
D = 128
NPAD = 10240
NC, NS = 2, 16
NW = NC * NS
CH = 80
LABP = 128

def _sc_compiler_params():
    cp = pltpu.CompilerParams()
    if "needs_layout_passes" in pltpu.CompilerParams.__dataclass_fields__:
        cp = dataclasses.replace(cp, needs_layout_passes=False)
    return cp


@functools.cache
def _mesh():
    return plsc.VectorSubcoreMesh(core_axis_name="c", subcore_axis_name="s",
                                  num_cores=NC, num_subcores=NS)


def _wid():
    return lax.axis_index("s") * NC + lax.axis_index("c")


def _gather_body(idx_hbm, feat_hbm, lab_hbm, feat_out, lab_out,
                 idxv, fbuf, lbuf, sem1, sem2):
    w = _wid()
    rows_per_tile = NPAD // NW

    @pl.loop(0, rows_per_tile // CH)
    def _chunk(c):
        base = w * rows_per_tile + c * CH
        pltpu.sync_copy(idx_hbm.at[pl.ds(base, CH)], idxv)
        cp1 = pltpu.async_copy(feat_hbm.at[idxv], fbuf, sem1)
        cp2 = pltpu.async_copy(lab_hbm.at[idxv], lbuf, sem2)
        cp1.wait()
        cp2.wait()
        pltpu.sync_copy(fbuf, feat_out.at[pl.ds(base, CH)])
        pltpu.sync_copy(lbuf, lab_out.at[pl.ds(base, CH)])


@functools.cache
def _gather_call():
    return pl.kernel(
        _gather_body,
        out_type=(jax.ShapeDtypeStruct((NPAD, D), jnp.float32),
                  jax.ShapeDtypeStruct((NPAD, LABP), jnp.float32)),
        mesh=_mesh(),
        scratch_types=[
            pltpu.VMEM((CH,), jnp.int32),
            pltpu.VMEM((CH, D), jnp.float32),
            pltpu.VMEM((CH, LABP), jnp.float32),
            pltpu.SemaphoreType.DMA,
            pltpu.SemaphoreType.DMA,
        ],
    )


def _zero_buf(rowsv):
    @pl.loop(0, CH)
    def _z(e):
        for j in range(D // 16):
            rowsv.at[e, pl.ds(j * 16, 16)][...] = jnp.zeros((16,), jnp.float32)


def _zero_acc(rowsv, acc):
    sid = lax.axis_index("s")
    rows_per_sub = NPAD // NS

    @pl.loop(0, rows_per_sub // CH)
    def _z(k):
        pltpu.sync_copy(rowsv, acc.at[pl.ds(sid * rows_per_sub + k * CH, CH)])


NBUF = 4
PFD = 2


def _scale_rows(ebuf, rowsv):
    @plsc.parallel_loop(0, CH, unroll=16)
    def _scale(e):
        vrow = jnp.full((16,), 2, dtype=jnp.int32)
        vidx = jnp.full((16,), e, dtype=jnp.int32)
        v = plsc.bitcast(plsc.load_gather(ebuf, [vrow, vidx]), jnp.float32)
        for j in range(D // 16):
            sl = pl.ds(j * 16, 16)
            rowsv.at[e, sl][...] = rowsv.at[e, sl][...] * v


def _process_chunk(x_hbm, ep_hbm, acc, bufs, c, b, pf_gc, gc_end):
    ebuf, rowsv, gsem, isem, ssem = bufs[b]
    pb = (b + PFD) % NBUF
    pebuf, prowsv, pgsem, pisem, pssem = bufs[pb]

    if pf_gc is not None:
        @pl.when(pf_gc < gc_end)
        def _pf_idx():
            @pl.when(c >= NBUF - PFD)
            def _w():
                pltpu.make_async_copy(prowsv, acc.at[pebuf.at[0]], pssem).wait()
            pltpu.async_copy(ep_hbm.at[pf_gc], pebuf, pisem)

    pltpu.make_async_copy(x_hbm.at[ebuf.at[1]], rowsv, gsem).wait()
    _scale_rows(ebuf, rowsv)
    pltpu.async_copy(rowsv, acc.at[ebuf.at[0]], ssem, add=True)

    if pf_gc is not None:
        @pl.when(pf_gc < gc_end)
        def _pf_gather():
            pltpu.make_async_copy(ep_hbm.at[pf_gc], pebuf, pisem).wait()
            pltpu.async_copy(x_hbm.at[pebuf.at[1]], prowsv, pgsem)


def _issue_first(x_hbm, ep_hbm, bufs, b, gc):
    ebuf, rowsv, gsem, isem, ssem = bufs[b]
    pltpu.sync_copy(ep_hbm.at[gc], ebuf)
    pltpu.async_copy(x_hbm.at[ebuf.at[1]], rowsv, gsem)


def _spmm_edges(x_hbm, ep_hbm, bufs, acc, cbase, nchunk):
    gc_end = cbase + nchunk
    for k in range(PFD):
        _issue_first(x_hbm, ep_hbm, bufs, k, cbase + k)
    plsc.subcore_barrier()

    @pl.loop(0, nchunk // NBUF)
    def _round(i):
        for b in range(NBUF):
            c = NBUF * i + b
            _process_chunk(x_hbm, ep_hbm, acc, bufs, c, b, cbase + c + PFD,
                           gc_end)
    for r in range(nchunk % NBUF):
        c = (nchunk // NBUF) * NBUF + r
        _process_chunk(x_hbm, ep_hbm, acc, bufs, c, c % NBUF, None, gc_end)

    for b in range(NBUF):
        ebuf, rowsv, gsem, isem, ssem = bufs[b]
        pltpu.make_async_copy(rowsv, acc.at[ebuf.at[0]], ssem).wait()


def _drain_acc(acc, out_hbm_core):
    sid = lax.axis_index("s")
    rows_per_sub = NPAD // NS

    @pl.loop(0, rows_per_sub // CH)
    def _d(k):
        r0 = sid * rows_per_sub + k * CH
        pltpu.sync_copy(acc.at[pl.ds(r0, CH)], out_hbm_core.at[pl.ds(r0, CH)])


_SPMM_SCRATCH = [
    s for _ in range(NBUF)
    for s in (pltpu.VMEM((3, CH), jnp.int32),
              pltpu.VMEM((CH, D), jnp.float32),
              pltpu.SemaphoreType.DMA,
              pltpu.SemaphoreType.DMA,
              pltpu.SemaphoreType.DMA)
] + [pltpu.VMEM_SHARED((NPAD, D), jnp.float32)]


def _split_scratch(rest):
    bufs = tuple(tuple(rest[5 * b:5 * b + 5]) for b in range(NBUF))
    return bufs, bufs[0][1], rest[5 * NBUF]


def _spmm1_body(x_hbm, ep_hbm, out_hbm, *rest):
    bufs, zbuf, acc = _split_scratch(rest)
    cid = lax.axis_index("c")
    cpt = (E // CH) // NW
    _zero_buf(zbuf)
    _zero_acc(zbuf, acc)
    _spmm_edges(x_hbm, ep_hbm, bufs, acc, _wid() * cpt, cpt)
    plsc.subcore_barrier()
    _drain_acc(acc, out_hbm.at[cid])


@functools.cache
def _spmm1_call():
    return pl.kernel(
        _spmm1_body,
        out_type=jax.ShapeDtypeStruct((NC, NPAD, D), jnp.float32),
        mesh=_mesh(),
        scratch_types=_SPMM_SCRATCH,
        compiler_params=_sc_compiler_params(),
    )


def _spmm2_body(x0_hbm, x1_hbm, ep_hbm, out_hbm, *rest):
    bufs, zbuf, acc = _split_scratch(rest)
    cid = lax.axis_index("c")
    sid = lax.axis_index("s")
    cpt = (E // CH) // NS
    _zero_buf(zbuf)
    _zero_acc(zbuf, acc)

    @pl.when(cid == 0)
    def _c0():
        _spmm_edges(x0_hbm, ep_hbm, bufs, acc, sid * cpt, cpt)

    @pl.when(cid == 1)
    def _c1():
        _spmm_edges(x1_hbm, ep_hbm, bufs, acc, sid * cpt, cpt)

    plsc.subcore_barrier()
    _drain_acc(acc, out_hbm.at[cid])


@functools.cache
def _spmm2_call():
    return pl.kernel(
        _spmm2_body,
        out_type=jax.ShapeDtypeStruct((NC, NPAD, D), jnp.float32),
        mesh=_mesh(),
        scratch_types=_SPMM_SCRATCH,
        compiler_params=_sc_compiler_params(),
    )


_BM = 2048


def _layer1_body(feat_ref, ax0_ref, ax1_ref, w00_ref, w01_ref, b00_ref,
                 b01_ref, p0_ref, p1_ref):
    f = feat_ref[...]
    ax = ax0_ref[...] + ax1_ref[...]
    p0_ref[...] = jnp.maximum(
        jnp.dot(f, w00_ref[...], preferred_element_type=jnp.float32,
                precision=jax.lax.Precision.HIGHEST)
        + b00_ref[...], 0.0)
    p1_ref[...] = jnp.maximum(
        jnp.dot(ax, w01_ref[...], preferred_element_type=jnp.float32,
                precision=jax.lax.Precision.HIGHEST)
        + b01_ref[...], 0.0)


def _layer1_call(feat, ax0, ax1, w00, w01, b00, b01):
    row_spec = pl.BlockSpec((_BM, D), lambda i: (i, 0))
    full = pl.BlockSpec((D, D), lambda i: (0, 0))
    bias = pl.BlockSpec((1, D), lambda i: (0, 0))
    return pl.pallas_call(
        _layer1_body,
        grid=(NPAD // _BM,),
        in_specs=[row_spec, row_spec, row_spec, full, full, bias, bias],
        out_specs=[row_spec, row_spec],
        out_shape=[jax.ShapeDtypeStruct((NPAD, D), jnp.float32)] * 2,
    )(feat, ax0, ax1, w00, w01, b00, b01)


def _layer2_body(p0_ref, p1_ref, a0_ref, a1_ref, w10a_ref, w10b_ref,
                 w11a_ref, w11b_ref, b10_ref, b11_ref, wc0_ref, wc1_ref,
                 bc_ref, pred_ref):
    p0, p1 = p0_ref[...], p1_ref[...]
    q0 = jnp.maximum(
        jnp.dot(p0, w10a_ref[...], preferred_element_type=jnp.float32,
                precision=jax.lax.Precision.HIGHEST)
        + jnp.dot(p1, w10b_ref[...], preferred_element_type=jnp.float32,
                precision=jax.lax.Precision.HIGHEST)
        + b10_ref[...], 0.0)
    q1 = jnp.maximum(
        jnp.dot(a0_ref[...], w11a_ref[...], preferred_element_type=jnp.float32,
                precision=jax.lax.Precision.HIGHEST)
        + jnp.dot(a1_ref[...], w11b_ref[...], preferred_element_type=jnp.float32,
                precision=jax.lax.Precision.HIGHEST)
        + b11_ref[...], 0.0)
    s = jnp.sum(q0 * q0, axis=1, keepdims=True) + jnp.sum(q1 * q1, axis=1,
                                                          keepdims=True)
    norm = jnp.maximum(jnp.sqrt(s), 1e-12)
    pred_ref[...] = (
        jnp.dot(q0, wc0_ref[...], preferred_element_type=jnp.float32,
                precision=jax.lax.Precision.HIGHEST)
        + jnp.dot(q1, wc1_ref[...], preferred_element_type=jnp.float32,
                precision=jax.lax.Precision.HIGHEST)
    ) / norm + bc_ref[...]


def _layer2_call(p0, p1, a0, a1, w10a, w10b, w11a, w11b, b10, b11, wc0, wc1,
                 bcp):
    row_spec = pl.BlockSpec((_BM, D), lambda i: (i, 0))
    full = pl.BlockSpec((D, D), lambda i: (0, 0))
    bias = pl.BlockSpec((1, D), lambda i: (0, 0))
    wc_spec = pl.BlockSpec((D, LABP), lambda i: (0, 0))
    bc_spec = pl.BlockSpec((1, LABP), lambda i: (0, 0))
    return pl.pallas_call(
        _layer2_body,
        grid=(NPAD // _BM,),
        in_specs=[row_spec, row_spec, row_spec, row_spec, full, full, full,
                  full, bias, bias, wc_spec, wc_spec, bc_spec],
        out_specs=pl.BlockSpec((_BM, LABP), lambda i: (i, 0)),
        out_shape=jax.ShapeDtypeStruct((NPAD, LABP), jnp.float32),
    )(p0, p1, a0, a1, w10a, w10b, w11a, w11b, b10, b11, wc0, wc1, bcp)


def _labpad_body(lab_ref, o_ref):
    o_ref[...] = jnp.concatenate(
        [lab_ref[...],
         jnp.full((lab_ref.shape[0], LABP - lab_ref.shape[1]), -1e30,
                  jnp.float32)], axis=1)


def _labpad_call(label_full):
    c = label_full.shape[1]
    bm = 2000
    return pl.pallas_call(
        _labpad_body,
        grid=(N_FULL // bm,),
        in_specs=[pl.BlockSpec((bm, c), lambda i: (i, 0))],
        out_specs=pl.BlockSpec((bm, LABP), lambda i: (i, 0)),
        out_shape=jax.ShapeDtypeStruct((N_FULL, LABP), jnp.float32),
    )(label_full)


def _argmax_body(lab_ref, o_ref):
    x = lab_ref[...]
    m = jnp.max(x, axis=1, keepdims=True)
    col = jax.lax.broadcasted_iota(jnp.int32, x.shape, 1)
    idx = jnp.where(x == m, col, jnp.int32(LABP))
    o_ref[...] = jnp.min(idx, axis=1).astype(jnp.int32)[:, None]


def _argmax_call(labg):
    return pl.pallas_call(
        _argmax_body,
        grid=(NPAD // _BM,),
        in_specs=[pl.BlockSpec((_BM, LABP), lambda i: (i, 0))],
        out_specs=pl.BlockSpec((_BM, 1), lambda i: (i, 0)),
        out_shape=jax.ShapeDtypeStruct((NPAD, 1), jnp.int32),
    )(labg)


def kernel(node_subgraph, adj_row, adj_col, adj_val, feat_full, label_full,
           W0_0, W0_1, b0_0, b0_1, W1_0, W1_1, b1_0, b1_1, Wc, bc):
    C = Wc.shape[1]
    ns_pad = jnp.concatenate(
        [node_subgraph, jnp.zeros((NPAD - N_SUB,), jnp.int32)])
    lab_pad = _labpad_call(label_full)
    epack = jnp.stack(
        [adj_row.reshape(E // CH, CH),
         adj_col.reshape(E // CH, CH),
         jax.lax.bitcast_convert_type(adj_val, jnp.int32).reshape(E // CH, CH)],
        axis=1)

    featg, labg = _gather_call()(ns_pad, feat_full, lab_pad)

    ax = _spmm1_call()(featg, epack)
    p0, p1 = _layer1_call(featg, ax[0], ax[1], W0_0, W0_1,
                          b0_0[None, :], b0_1[None, :])
    ah = _spmm2_call()(p0, p1, epack)
    pred_pad = _layer2_call(
        p0, p1, ah[0], ah[1],
        W1_0[:D], W1_0[D:], W1_1[:D], W1_1[D:],
        b1_0[None, :], b1_1[None, :],
        jnp.pad(Wc[:D], ((0, 0), (0, LABP - C))),
        jnp.pad(Wc[D:], ((0, 0), (0, LABP - C))),
        jnp.pad(bc, (0, LABP - C))[None, :])
    conv = _argmax_call(labg)

    return (pred_pad[:N_SUB, :C], labg[:N_SUB, :C], conv[:N_SUB, 0])
